# baseline (speedup 1.0000x reference)
import functools

import jax
import jax.numpy as jnp
from jax import lax
from jax.experimental import pallas as pl
from jax.experimental.pallas import tpu as pltpu
from jax.experimental.pallas import tpu_sc as plsc

N = 10000
E = 320000
D = 128
NC, NS = 2, 16
NW = NC * NS
CSZ = 128
CHUNKS = 2 * -(-(-(-E // NW)) // (2 * CSZ))
EPAD = NW * CHUNKS * CSZ
HCH = CHUNKS // 2
NROW = 10240
RPS = NROW // NS
CW = 8

def _sc_agg_counts_body(y_hbm, src_hbm, dst_hbm, z_hbm, zc_hbm, ones_hbm,
                        p_hbm, cnt_hbm,
                        src_v, dst_v, rows0_v, rows1_v, ones_v, acc_sh, cnt_sh,
                        gsem0, gsem1, ssem0, ssem1, csem0, csem1):
    rows = (rows0_v, rows1_v)
    gsem = (gsem0, gsem1)
    ssem = (ssem0, ssem1)
    csem = (csem0, csem1)
    c = lax.axis_index("c")
    s = lax.axis_index("s")
    w = s * NC + c
    pltpu.sync_copy(z_hbm.at[pl.ds(s * RPS, RPS)],
                    acc_sh.at[pl.ds(s * RPS, RPS)])
    pltpu.sync_copy(zc_hbm.at[pl.ds(s * RPS, RPS)],
                    cnt_sh.at[pl.ds(s * RPS, RPS)])
    pltpu.sync_copy(ones_hbm, ones_v)
    plsc.subcore_barrier()
    for h in (0, 1):
        pltpu.sync_copy(src_hbm.at[w].at[pl.ds(h * HCH, HCH)], src_v)
        pltpu.sync_copy(dst_hbm.at[w].at[pl.ds(h * HCH, HCH)], dst_v)
        pltpu.async_copy(y_hbm.at[src_v.at[0]], rows[0], gsem[0])
        pltpu.async_copy(y_hbm.at[src_v.at[1]], rows[1], gsem[1])

        def step(i, carry):
            j = 2 * i
            pltpu.make_async_copy(y_hbm.at[src_v.at[0]], rows[0],
                                  gsem[0]).wait()
            pltpu.make_async_copy(y_hbm.at[src_v.at[0]], rows[1],
                                  gsem[1]).wait()
            s0 = pltpu.async_copy(rows[0], acc_sh.at[dst_v.at[j]],
                                  ssem[0], add=True)
            s1 = pltpu.async_copy(rows[1], acc_sh.at[dst_v.at[j + 1]],
                                  ssem[1], add=True)
            c0 = pltpu.async_copy(ones_v, cnt_sh.at[dst_v.at[j]],
                                  csem[0], add=True)
            c1 = pltpu.async_copy(ones_v, cnt_sh.at[dst_v.at[j + 1]],
                                  csem[1], add=True)
            s0.wait()
            c0.wait()

            @pl.when(j + 2 < HCH)
            def _prefetch0():
                pltpu.async_copy(y_hbm.at[src_v.at[j + 2]], rows[0], gsem[0])

            s1.wait()
            c1.wait()

            @pl.when(j + 3 < HCH)
            def _prefetch1():
                pltpu.async_copy(y_hbm.at[src_v.at[j + 3]], rows[1], gsem[1])
            return carry

        lax.fori_loop(0, HCH // 2, step, 0)
    plsc.subcore_barrier()
    pltpu.sync_copy(acc_sh.at[pl.ds(s * RPS, RPS)],
                    p_hbm.at[c].at[pl.ds(s * RPS, RPS)])
    pltpu.sync_copy(cnt_sh.at[pl.ds(s * RPS, RPS)],
                    cnt_hbm.at[c].at[pl.ds(s * RPS, RPS)])


def _sc_agg_body(y_hbm, src_hbm, dst_hbm, z_hbm,
                 p_hbm,
                 src_v, dst_v, rows0_v, rows1_v, acc_sh,
                 gsem0, gsem1, ssem0, ssem1):
    rows = (rows0_v, rows1_v)
    gsem = (gsem0, gsem1)
    ssem = (ssem0, ssem1)
    c = lax.axis_index("c")
    s = lax.axis_index("s")
    w = s * NC + c
    pltpu.sync_copy(z_hbm.at[pl.ds(s * RPS, RPS)],
                    acc_sh.at[pl.ds(s * RPS, RPS)])
    plsc.subcore_barrier()
    for h in (0, 1):
        pltpu.sync_copy(src_hbm.at[w].at[pl.ds(h * HCH, HCH)], src_v)
        pltpu.sync_copy(dst_hbm.at[w].at[pl.ds(h * HCH, HCH)], dst_v)
        pltpu.async_copy(y_hbm.at[src_v.at[0]], rows[0], gsem[0])
        pltpu.async_copy(y_hbm.at[src_v.at[1]], rows[1], gsem[1])

        def step(i, carry):
            j = 2 * i
            pltpu.make_async_copy(y_hbm.at[src_v.at[0]], rows[0],
                                  gsem[0]).wait()
            pltpu.make_async_copy(y_hbm.at[src_v.at[0]], rows[1],
                                  gsem[1]).wait()
            s0 = pltpu.async_copy(rows[0], acc_sh.at[dst_v.at[j]],
                                  ssem[0], add=True)
            s1 = pltpu.async_copy(rows[1], acc_sh.at[dst_v.at[j + 1]],
                                  ssem[1], add=True)
            s0.wait()

            @pl.when(j + 2 < HCH)
            def _prefetch0():
                pltpu.async_copy(y_hbm.at[src_v.at[j + 2]], rows[0], gsem[0])

            s1.wait()

            @pl.when(j + 3 < HCH)
            def _prefetch1():
                pltpu.async_copy(y_hbm.at[src_v.at[j + 3]], rows[1], gsem[1])
            return carry

        lax.fori_loop(0, HCH // 2, step, 0)
    plsc.subcore_barrier()
    pltpu.sync_copy(acc_sh.at[pl.ds(s * RPS, RPS)],
                    p_hbm.at[c].at[pl.ds(s * RPS, RPS)])


@functools.cache
def _build_sc_kernels():
    mesh = plsc.VectorSubcoreMesh(core_axis_name="c", subcore_axis_name="s",
                                  num_cores=NC, num_subcores=NS)
    sc_agg_counts = pl.kernel(
        _sc_agg_counts_body,
        out_type=(jax.ShapeDtypeStruct((NC, NROW, D), jnp.float32),
                  jax.ShapeDtypeStruct((NC, NROW), jnp.float32)),
        mesh=mesh,
        scratch_types=[
            pltpu.VMEM((HCH, CSZ), jnp.int32),
            pltpu.VMEM((HCH, CSZ), jnp.int32),
            pltpu.VMEM((CSZ, D), jnp.float32),
            pltpu.VMEM((CSZ, D), jnp.float32),
            pltpu.VMEM((CSZ,), jnp.float32),
            pltpu.VMEM_SHARED((NROW, D), jnp.float32),
            pltpu.VMEM_SHARED((NROW,), jnp.float32),
        ] + [pltpu.SemaphoreType.DMA] * 6,
    )
    sc_agg = pl.kernel(
        _sc_agg_body,
        out_type=jax.ShapeDtypeStruct((NC, NROW, D), jnp.float32),
        mesh=mesh,
        scratch_types=[
            pltpu.VMEM((HCH, CSZ), jnp.int32),
            pltpu.VMEM((HCH, CSZ), jnp.int32),
            pltpu.VMEM((CSZ, D), jnp.float32),
            pltpu.VMEM((CSZ, D), jnp.float32),
            pltpu.VMEM_SHARED((NROW, D), jnp.float32),
        ] + [pltpu.SemaphoreType.DMA] * 4,
    )
    return sc_agg_counts, sc_agg

BR = 2048
_DOT_T = (((1,), (1,)), ((), ()))


def _proj_body(x_ref, w_ref, o_ref):
    o_ref[...] = lax.dot_general(x_ref[...], w_ref[...], _DOT_T,
                                 preferred_element_type=jnp.float32)


def _tc_proj(x, w):
    return pl.pallas_call(
        _proj_body,
        grid=(NROW // BR,),
        in_specs=[pl.BlockSpec((BR, D), lambda i: (i, 0)),
                  pl.BlockSpec((D, D), lambda i: (0, 0))],
        out_specs=pl.BlockSpec((BR, D), lambda i: (i, 0)),
        out_shape=jax.ShapeDtypeStruct((NROW, D), jnp.float32),
    )(x, w)


def _mid_body(p_ref, c_ref, x_ref, w1r, b1, w2l, h1_ref, y2_ref):
    cnt = c_ref[0] + c_ref[1]
    agg = (p_ref[0] + p_ref[1]) / jnp.maximum(cnt, 1.0)
    root = lax.dot_general(x_ref[...], w1r[...], _DOT_T,
                           preferred_element_type=jnp.float32)
    h1 = jnp.maximum(agg + root + b1[...], 0.0)
    h1_ref[...] = h1
    y2 = lax.dot_general(h1, w2l[...], _DOT_T,
                         preferred_element_type=jnp.float32)
    gid = pl.program_id(0) * BR + lax.broadcasted_iota(jnp.int32, (BR, 1), 0)
    y2_ref[...] = jnp.where(gid < N, y2, 0.0)


def _tc_mid(p, cnts, x, w1r, b1, w2l):
    blk = lambda i: (i, 0)
    blk3 = lambda i: (0, i, 0)
    whole = lambda i: (0, 0)
    return pl.pallas_call(
        _mid_body,
        grid=(NROW // BR,),
        in_specs=[pl.BlockSpec((NC, BR, D), blk3),
                  pl.BlockSpec((NC, BR, 1), blk3),
                  pl.BlockSpec((BR, D), blk),
                  pl.BlockSpec((D, D), whole), pl.BlockSpec((1, D), whole),
                  pl.BlockSpec((D, D), whole)],
        out_specs=[pl.BlockSpec((BR, D), blk), pl.BlockSpec((BR, D), blk)],
        out_shape=(jax.ShapeDtypeStruct((NROW, D), jnp.float32),
                   jax.ShapeDtypeStruct((NROW, D), jnp.float32)),
    )(p, cnts, x, w1r, b1, w2l)


def _fin_body(p_ref, c_ref, h1_ref, w2r, b2, wlin, blin, o_ref):
    cnt = c_ref[0] + c_ref[1]
    agg = (p_ref[0] + p_ref[1]) / jnp.maximum(cnt, 1.0)
    root = lax.dot_general(h1_ref[...], w2r[...], _DOT_T,
                           preferred_element_type=jnp.float32)
    h2 = jnp.maximum(agg + root + b2[...], 0.0)
    logits = lax.dot_general(h2, wlin[...], _DOT_T,
                             preferred_element_type=jnp.float32) + blin[...]
    col = lax.broadcasted_iota(jnp.int32, logits.shape, 1)
    logits = jnp.where(col < 2, logits, -1e30)
    m = jnp.max(logits, axis=1, keepdims=True)
    sh = logits - m
    lse = jnp.log(jnp.sum(jnp.exp(sh), axis=1, keepdims=True))
    o_ref[...] = lax.slice(sh - lse, (0, 0), (sh.shape[0], 2))


def _tc_fin(p, cnts, h1, w2r, b2, wlin, blin):
    blk = lambda i: (i, 0)
    blk3 = lambda i: (0, i, 0)
    whole = lambda i: (0, 0)
    return pl.pallas_call(
        _fin_body,
        grid=(NROW // BR,),
        in_specs=[pl.BlockSpec((NC, BR, D), blk3),
                  pl.BlockSpec((NC, BR, 1), blk3),
                  pl.BlockSpec((BR, D), blk),
                  pl.BlockSpec((D, D), whole), pl.BlockSpec((1, D), whole),
                  pl.BlockSpec((CW, D), whole), pl.BlockSpec((1, CW), whole)],
        out_specs=pl.BlockSpec((BR, 2), blk),
        out_shape=jax.ShapeDtypeStruct((NROW, 2), jnp.float32),
    )(p, cnts, h1, w2r, b2, wlin, blin)


def kernel(x, edge_index, W1l, b1, W1r, W2l, b2, W2r, Wlin, blin):
    f32 = jnp.float32
    src = edge_index[0].astype(jnp.int32)
    dst = edge_index[1].astype(jnp.int32)
    padv = N + (jnp.arange(EPAD - E, dtype=jnp.int32) % (NROW - N))
    src3 = jnp.concatenate([src, padv]).reshape(NW, CHUNKS, CSZ)
    dst3 = jnp.concatenate([dst, padv]).reshape(NW, CHUNKS, CSZ)
    zrow = jnp.zeros((NROW, D), f32)
    zcnt = jnp.zeros((NROW,), f32)
    ones = jnp.ones((CSZ,), f32)
    sc_agg_counts, sc_agg = _build_sc_kernels()

    x_pad = jnp.concatenate([x, jnp.zeros((NROW - N, D), f32)], axis=0)
    y1 = _tc_proj(x_pad, W1l)
    P1, C1 = sc_agg_counts(y1, src3, dst3, zrow, zcnt, ones)
    cnts = C1.reshape(NC, NROW, 1)
    h1, y2 = _tc_mid(P1, cnts, x_pad, W1r, b1.reshape(1, D), W2l)
    P2 = sc_agg(y2, src3, dst3, zrow)
    wlin_pad = jnp.concatenate([Wlin, jnp.zeros((CW - 2, D), f32)], axis=0)
    blin_pad = jnp.concatenate([blin, jnp.zeros((CW - 2,), f32)]).reshape(1, CW)
    out_pad = _tc_fin(P2, cnts, h1, W2r, b2.reshape(1, D), wlin_pad, blin_pad)
    return out_pad[:N]


# speedup vs baseline: 1.2731x; 1.2731x over previous
import functools

import jax
import jax.numpy as jnp
from jax import lax
from jax.experimental import pallas as pl
from jax.experimental.pallas import tpu as pltpu
from jax.experimental.pallas import tpu_sc as plsc

N = 10000
E = 320000
D = 128
NC, NS = 2, 16
NW = NC * NS
CSZ = 128
CHUNKS = 2 * -(-(-(-E // NW)) // (2 * CSZ))
EPAD = NW * CHUNKS * CSZ
HCH = CHUNKS // 2
NROW = 10240
RPS = NROW // NS
CW = 8

def _sc_agg_counts_body(y_hbm, src_hbm, dst_hbm, z_hbm, zc_hbm, ones_hbm,
                        p_hbm, cnt_hbm,
                        src_v, dst_v, rows0_v, rows1_v, ones_v, acc_sh, cnt_sh,
                        gsem0, gsem1, ssem0, ssem1, csem0, csem1):
    rows = (rows0_v, rows1_v)
    gsem = (gsem0, gsem1)
    ssem = (ssem0, ssem1)
    csem = (csem0, csem1)
    c = lax.axis_index("c")
    s = lax.axis_index("s")
    w = s * NC + c
    pltpu.sync_copy(z_hbm.at[pl.ds(s * RPS, RPS)],
                    acc_sh.at[pl.ds(s * RPS, RPS)])
    pltpu.sync_copy(zc_hbm.at[pl.ds(s * RPS, RPS)],
                    cnt_sh.at[pl.ds(s * RPS, RPS)])
    pltpu.sync_copy(ones_hbm, ones_v)
    plsc.subcore_barrier()
    for h in (0, 1):
        pltpu.sync_copy(src_hbm.at[w].at[pl.ds(h * HCH, HCH)], src_v)
        pltpu.sync_copy(dst_hbm.at[w].at[pl.ds(h * HCH, HCH)], dst_v)
        pltpu.async_copy(y_hbm.at[src_v.at[0]], rows[0], gsem[0])
        pltpu.async_copy(y_hbm.at[src_v.at[1]], rows[1], gsem[1])

        def step(i, carry):
            for b in (0, 1):
                j = 2 * i + b
                pltpu.make_async_copy(y_hbm.at[src_v.at[0]], rows[b],
                                      gsem[b]).wait()
                s = pltpu.async_copy(rows[b], acc_sh.at[dst_v.at[j]],
                                     ssem[b], add=True)
                cc = pltpu.async_copy(ones_v, cnt_sh.at[dst_v.at[j]],
                                      csem[b], add=True)
                s.wait()
                cc.wait()

                @pl.when(j + 2 < HCH)
                def _prefetch():
                    pltpu.async_copy(y_hbm.at[src_v.at[j + 2]], rows[b],
                                     gsem[b])
            return carry

        lax.fori_loop(0, HCH // 2, step, 0)
    plsc.subcore_barrier()
    pltpu.sync_copy(acc_sh.at[pl.ds(s * RPS, RPS)],
                    p_hbm.at[c].at[pl.ds(s * RPS, RPS)])
    pltpu.sync_copy(cnt_sh.at[pl.ds(s * RPS, RPS)],
                    cnt_hbm.at[c].at[pl.ds(s * RPS, RPS)])


def _sc_agg_body(y_hbm, src_hbm, dst_hbm, z_hbm,
                 p_hbm,
                 src_v, dst_v, rows0_v, rows1_v, acc_sh,
                 gsem0, gsem1, ssem0, ssem1):
    rows = (rows0_v, rows1_v)
    gsem = (gsem0, gsem1)
    ssem = (ssem0, ssem1)
    c = lax.axis_index("c")
    s = lax.axis_index("s")
    w = s * NC + c
    pltpu.sync_copy(z_hbm.at[pl.ds(s * RPS, RPS)],
                    acc_sh.at[pl.ds(s * RPS, RPS)])
    plsc.subcore_barrier()
    for h in (0, 1):
        pltpu.sync_copy(src_hbm.at[w].at[pl.ds(h * HCH, HCH)], src_v)
        pltpu.sync_copy(dst_hbm.at[w].at[pl.ds(h * HCH, HCH)], dst_v)
        pltpu.async_copy(y_hbm.at[src_v.at[0]], rows[0], gsem[0])
        pltpu.async_copy(y_hbm.at[src_v.at[1]], rows[1], gsem[1])

        def step(i, carry):
            for b in (0, 1):
                j = 2 * i + b
                pltpu.make_async_copy(y_hbm.at[src_v.at[0]], rows[b],
                                      gsem[b]).wait()
                s = pltpu.async_copy(rows[b], acc_sh.at[dst_v.at[j]],
                                     ssem[b], add=True)
                s.wait()

                @pl.when(j + 2 < HCH)
                def _prefetch():
                    pltpu.async_copy(y_hbm.at[src_v.at[j + 2]], rows[b],
                                     gsem[b])
            return carry

        lax.fori_loop(0, HCH // 2, step, 0)
    plsc.subcore_barrier()
    pltpu.sync_copy(acc_sh.at[pl.ds(s * RPS, RPS)],
                    p_hbm.at[c].at[pl.ds(s * RPS, RPS)])


@functools.cache
def _build_sc_kernels():
    mesh = plsc.VectorSubcoreMesh(core_axis_name="c", subcore_axis_name="s",
                                  num_cores=NC, num_subcores=NS)
    sc_agg_counts = pl.kernel(
        _sc_agg_counts_body,
        out_type=(jax.ShapeDtypeStruct((NC, NROW, D), jnp.float32),
                  jax.ShapeDtypeStruct((NC, NROW), jnp.float32)),
        mesh=mesh,
        scratch_types=[
            pltpu.VMEM((HCH, CSZ), jnp.int32),
            pltpu.VMEM((HCH, CSZ), jnp.int32),
            pltpu.VMEM((CSZ, D), jnp.float32),
            pltpu.VMEM((CSZ, D), jnp.float32),
            pltpu.VMEM((CSZ,), jnp.float32),
            pltpu.VMEM_SHARED((NROW, D), jnp.float32),
            pltpu.VMEM_SHARED((NROW,), jnp.float32),
        ] + [pltpu.SemaphoreType.DMA] * 6,
    )
    sc_agg = pl.kernel(
        _sc_agg_body,
        out_type=jax.ShapeDtypeStruct((NC, NROW, D), jnp.float32),
        mesh=mesh,
        scratch_types=[
            pltpu.VMEM((HCH, CSZ), jnp.int32),
            pltpu.VMEM((HCH, CSZ), jnp.int32),
            pltpu.VMEM((CSZ, D), jnp.float32),
            pltpu.VMEM((CSZ, D), jnp.float32),
            pltpu.VMEM_SHARED((NROW, D), jnp.float32),
        ] + [pltpu.SemaphoreType.DMA] * 4,
    )
    return sc_agg_counts, sc_agg

BR = 2048
_DOT_T = (((1,), (1,)), ((), ()))


def _proj_body(x_ref, w_ref, o_ref):
    o_ref[...] = lax.dot_general(x_ref[...], w_ref[...], _DOT_T,
                                 preferred_element_type=jnp.float32)


def _tc_proj(x, w):
    return pl.pallas_call(
        _proj_body,
        grid=(NROW // BR,),
        in_specs=[pl.BlockSpec((BR, D), lambda i: (i, 0)),
                  pl.BlockSpec((D, D), lambda i: (0, 0))],
        out_specs=pl.BlockSpec((BR, D), lambda i: (i, 0)),
        out_shape=jax.ShapeDtypeStruct((NROW, D), jnp.float32),
    )(x, w)


def _mid_body(p_ref, c_ref, x_ref, w1r, b1, w2l, h1_ref, y2_ref):
    cnt = c_ref[0] + c_ref[1]
    agg = (p_ref[0] + p_ref[1]) / jnp.maximum(cnt, 1.0)
    root = lax.dot_general(x_ref[...], w1r[...], _DOT_T,
                           preferred_element_type=jnp.float32)
    h1 = jnp.maximum(agg + root + b1[...], 0.0)
    h1_ref[...] = h1
    y2 = lax.dot_general(h1, w2l[...], _DOT_T,
                         preferred_element_type=jnp.float32)
    gid = pl.program_id(0) * BR + lax.broadcasted_iota(jnp.int32, (BR, 1), 0)
    y2_ref[...] = jnp.where(gid < N, y2, 0.0)


def _tc_mid(p, cnts, x, w1r, b1, w2l):
    blk = lambda i: (i, 0)
    blk3 = lambda i: (0, i, 0)
    whole = lambda i: (0, 0)
    return pl.pallas_call(
        _mid_body,
        grid=(NROW // BR,),
        in_specs=[pl.BlockSpec((NC, BR, D), blk3),
                  pl.BlockSpec((NC, BR, 1), blk3),
                  pl.BlockSpec((BR, D), blk),
                  pl.BlockSpec((D, D), whole), pl.BlockSpec((1, D), whole),
                  pl.BlockSpec((D, D), whole)],
        out_specs=[pl.BlockSpec((BR, D), blk), pl.BlockSpec((BR, D), blk)],
        out_shape=(jax.ShapeDtypeStruct((NROW, D), jnp.float32),
                   jax.ShapeDtypeStruct((NROW, D), jnp.float32)),
    )(p, cnts, x, w1r, b1, w2l)


def _fin_body(p_ref, c_ref, h1_ref, w2r, b2, wlin, blin, o_ref):
    cnt = c_ref[0] + c_ref[1]
    agg = (p_ref[0] + p_ref[1]) / jnp.maximum(cnt, 1.0)
    root = lax.dot_general(h1_ref[...], w2r[...], _DOT_T,
                           preferred_element_type=jnp.float32)
    h2 = jnp.maximum(agg + root + b2[...], 0.0)
    logits = lax.dot_general(h2, wlin[...], _DOT_T,
                             preferred_element_type=jnp.float32) + blin[...]
    col = lax.broadcasted_iota(jnp.int32, logits.shape, 1)
    logits = jnp.where(col < 2, logits, -1e30)
    m = jnp.max(logits, axis=1, keepdims=True)
    sh = logits - m
    lse = jnp.log(jnp.sum(jnp.exp(sh), axis=1, keepdims=True))
    o_ref[...] = lax.slice(sh - lse, (0, 0), (sh.shape[0], 2))


def _tc_fin(p, cnts, h1, w2r, b2, wlin, blin):
    blk = lambda i: (i, 0)
    blk3 = lambda i: (0, i, 0)
    whole = lambda i: (0, 0)
    return pl.pallas_call(
        _fin_body,
        grid=(NROW // BR,),
        in_specs=[pl.BlockSpec((NC, BR, D), blk3),
                  pl.BlockSpec((NC, BR, 1), blk3),
                  pl.BlockSpec((BR, D), blk),
                  pl.BlockSpec((D, D), whole), pl.BlockSpec((1, D), whole),
                  pl.BlockSpec((CW, D), whole), pl.BlockSpec((1, CW), whole)],
        out_specs=pl.BlockSpec((BR, 2), blk),
        out_shape=jax.ShapeDtypeStruct((N, 2), jnp.float32),
    )(p, cnts, h1, w2r, b2, wlin, blin)


def kernel(x, edge_index, W1l, b1, W1r, W2l, b2, W2r, Wlin, blin):
    f32 = jnp.float32
    src = edge_index[0].astype(jnp.int32)
    dst = edge_index[1].astype(jnp.int32)
    padv = N + (jnp.arange(EPAD - E, dtype=jnp.int32) % (NROW - N))
    src3 = jnp.concatenate([src, padv]).reshape(NW, CHUNKS, CSZ)
    dst3 = jnp.concatenate([dst, padv]).reshape(NW, CHUNKS, CSZ)
    zrow = jnp.zeros((NROW, D), f32)
    zcnt = jnp.zeros((NROW,), f32)
    ones = jnp.ones((CSZ,), f32)
    sc_agg_counts, sc_agg = _build_sc_kernels()

    x_pad = jnp.concatenate([x, jnp.zeros((NROW - N, D), f32)], axis=0)
    y1 = _tc_proj(x_pad, W1l)
    P1, C1 = sc_agg_counts(y1, src3, dst3, zrow, zcnt, ones)
    cnts = C1.reshape(NC, NROW, 1)
    h1, y2 = _tc_mid(P1, cnts, x_pad, W1r, b1.reshape(1, D), W2l)
    P2 = sc_agg(y2, src3, dst3, zrow)
    wlin_pad = jnp.concatenate([Wlin, jnp.zeros((CW - 2, D), f32)], axis=0)
    blin_pad = jnp.concatenate([blin, jnp.zeros((CW - 2,), f32)]).reshape(1, CW)
    return _tc_fin(P2, cnts, h1, W2r, b2.reshape(1, D), wlin_pad, blin_pad)


# speedup vs baseline: 1.2783x; 1.0041x over previous
import functools

import jax
import jax.numpy as jnp
from jax import lax
from jax.experimental import pallas as pl
from jax.experimental.pallas import tpu as pltpu
from jax.experimental.pallas import tpu_sc as plsc

N = 10000
E = 320000
D = 128
NC, NS = 2, 16
NW = NC * NS
CSZ = 128
CHUNKS = 2 * -(-(-(-E // NW)) // (2 * CSZ))
EPAD = NW * CHUNKS * CSZ
HCH = CHUNKS // 2
NROW = 10240
RPS = NROW // NS
CW = 8

def _sc_agg_counts_body(y_hbm, src_hbm, dst_hbm, z_hbm, zc_hbm, ones_hbm,
                        p_hbm, cnt_hbm,
                        src_v, dst_v, rows0_v, rows1_v, ones_v, acc_sh, cnt_sh,
                        gsem0, gsem1, ssem0, ssem1, csem0, csem1):
    rows = (rows0_v, rows1_v)
    gsem = (gsem0, gsem1)
    ssem = (ssem0, ssem1)
    csem = (csem0, csem1)
    c = lax.axis_index("c")
    s = lax.axis_index("s")
    w = s * NC + c
    pltpu.sync_copy(z_hbm.at[pl.ds(s * RPS, RPS)],
                    acc_sh.at[pl.ds(s * RPS, RPS)])
    pltpu.sync_copy(zc_hbm.at[pl.ds(s * RPS, RPS)],
                    cnt_sh.at[pl.ds(s * RPS, RPS)])
    pltpu.sync_copy(ones_hbm, ones_v)
    plsc.subcore_barrier()
    for h in (0, 1):
        pltpu.sync_copy(src_hbm.at[w].at[pl.ds(h * HCH, HCH)], src_v)
        pltpu.sync_copy(dst_hbm.at[w].at[pl.ds(h * HCH, HCH)], dst_v)
        pltpu.async_copy(y_hbm.at[src_v.at[0]], rows[0], gsem[0])
        pltpu.async_copy(y_hbm.at[src_v.at[1]], rows[1], gsem[1])

        def step(i, carry):
            for b in (0, 1):
                j = 2 * i + b
                pltpu.make_async_copy(y_hbm.at[src_v.at[0]], rows[b],
                                      gsem[b]).wait()
                cc = pltpu.async_copy(ones_v, cnt_sh.at[dst_v.at[j]],
                                      csem[b], add=True)
                s = pltpu.async_copy(rows[b], acc_sh.at[dst_v.at[j]],
                                     ssem[b], add=True)
                s.wait()

                @pl.when(j + 2 < HCH)
                def _prefetch():
                    pltpu.async_copy(y_hbm.at[src_v.at[j + 2]], rows[b],
                                     gsem[b])

                cc.wait()
            return carry

        lax.fori_loop(0, HCH // 2, step, 0)
    plsc.subcore_barrier()
    pltpu.sync_copy(acc_sh.at[pl.ds(s * RPS, RPS)],
                    p_hbm.at[c].at[pl.ds(s * RPS, RPS)])
    pltpu.sync_copy(cnt_sh.at[pl.ds(s * RPS, RPS)],
                    cnt_hbm.at[c].at[pl.ds(s * RPS, RPS)])


def _sc_agg_body(y_hbm, src_hbm, dst_hbm, z_hbm,
                 p_hbm,
                 src_v, dst_v, rows0_v, rows1_v, acc_sh,
                 gsem0, gsem1, ssem0, ssem1):
    rows = (rows0_v, rows1_v)
    gsem = (gsem0, gsem1)
    ssem = (ssem0, ssem1)
    c = lax.axis_index("c")
    s = lax.axis_index("s")
    w = s * NC + c
    pltpu.sync_copy(z_hbm.at[pl.ds(s * RPS, RPS)],
                    acc_sh.at[pl.ds(s * RPS, RPS)])
    plsc.subcore_barrier()
    for h in (0, 1):
        pltpu.sync_copy(src_hbm.at[w].at[pl.ds(h * HCH, HCH)], src_v)
        pltpu.sync_copy(dst_hbm.at[w].at[pl.ds(h * HCH, HCH)], dst_v)
        pltpu.async_copy(y_hbm.at[src_v.at[0]], rows[0], gsem[0])
        pltpu.async_copy(y_hbm.at[src_v.at[1]], rows[1], gsem[1])

        def step(i, carry):
            for b in (0, 1):
                j = 2 * i + b
                pltpu.make_async_copy(y_hbm.at[src_v.at[0]], rows[b],
                                      gsem[b]).wait()
                s = pltpu.async_copy(rows[b], acc_sh.at[dst_v.at[j]],
                                     ssem[b], add=True)
                s.wait()

                @pl.when(j + 2 < HCH)
                def _prefetch():
                    pltpu.async_copy(y_hbm.at[src_v.at[j + 2]], rows[b],
                                     gsem[b])
            return carry

        lax.fori_loop(0, HCH // 2, step, 0)
    plsc.subcore_barrier()
    pltpu.sync_copy(acc_sh.at[pl.ds(s * RPS, RPS)],
                    p_hbm.at[c].at[pl.ds(s * RPS, RPS)])


@functools.cache
def _build_sc_kernels():
    mesh = plsc.VectorSubcoreMesh(core_axis_name="c", subcore_axis_name="s",
                                  num_cores=NC, num_subcores=NS)
    sc_agg_counts = pl.kernel(
        _sc_agg_counts_body,
        out_type=(jax.ShapeDtypeStruct((NC, NROW, D), jnp.float32),
                  jax.ShapeDtypeStruct((NC, NROW), jnp.float32)),
        mesh=mesh,
        scratch_types=[
            pltpu.VMEM((HCH, CSZ), jnp.int32),
            pltpu.VMEM((HCH, CSZ), jnp.int32),
            pltpu.VMEM((CSZ, D), jnp.float32),
            pltpu.VMEM((CSZ, D), jnp.float32),
            pltpu.VMEM((CSZ,), jnp.float32),
            pltpu.VMEM_SHARED((NROW, D), jnp.float32),
            pltpu.VMEM_SHARED((NROW,), jnp.float32),
        ] + [pltpu.SemaphoreType.DMA] * 6,
    )
    sc_agg = pl.kernel(
        _sc_agg_body,
        out_type=jax.ShapeDtypeStruct((NC, NROW, D), jnp.float32),
        mesh=mesh,
        scratch_types=[
            pltpu.VMEM((HCH, CSZ), jnp.int32),
            pltpu.VMEM((HCH, CSZ), jnp.int32),
            pltpu.VMEM((CSZ, D), jnp.float32),
            pltpu.VMEM((CSZ, D), jnp.float32),
            pltpu.VMEM_SHARED((NROW, D), jnp.float32),
        ] + [pltpu.SemaphoreType.DMA] * 4,
    )
    return sc_agg_counts, sc_agg

BR = 2048
_DOT_T = (((1,), (1,)), ((), ()))


def _proj_body(x_ref, w_ref, o_ref):
    o_ref[...] = lax.dot_general(x_ref[...], w_ref[...], _DOT_T,
                                 preferred_element_type=jnp.float32)


def _tc_proj(x, w):
    return pl.pallas_call(
        _proj_body,
        grid=(NROW // BR,),
        in_specs=[pl.BlockSpec((BR, D), lambda i: (i, 0)),
                  pl.BlockSpec((D, D), lambda i: (0, 0))],
        out_specs=pl.BlockSpec((BR, D), lambda i: (i, 0)),
        out_shape=jax.ShapeDtypeStruct((NROW, D), jnp.float32),
    )(x, w)


def _mid_body(p_ref, c_ref, x_ref, w1r, b1, w2l, h1_ref, y2_ref):
    cnt = c_ref[0] + c_ref[1]
    agg = (p_ref[0] + p_ref[1]) / jnp.maximum(cnt, 1.0)
    root = lax.dot_general(x_ref[...], w1r[...], _DOT_T,
                           preferred_element_type=jnp.float32)
    h1 = jnp.maximum(agg + root + b1[...], 0.0)
    h1_ref[...] = h1
    y2 = lax.dot_general(h1, w2l[...], _DOT_T,
                         preferred_element_type=jnp.float32)
    gid = pl.program_id(0) * BR + lax.broadcasted_iota(jnp.int32, (BR, 1), 0)
    y2_ref[...] = jnp.where(gid < N, y2, 0.0)


def _tc_mid(p, cnts, x, w1r, b1, w2l):
    blk = lambda i: (i, 0)
    blk3 = lambda i: (0, i, 0)
    whole = lambda i: (0, 0)
    return pl.pallas_call(
        _mid_body,
        grid=(NROW // BR,),
        in_specs=[pl.BlockSpec((NC, BR, D), blk3),
                  pl.BlockSpec((NC, BR, 1), blk3),
                  pl.BlockSpec((BR, D), blk),
                  pl.BlockSpec((D, D), whole), pl.BlockSpec((1, D), whole),
                  pl.BlockSpec((D, D), whole)],
        out_specs=[pl.BlockSpec((BR, D), blk), pl.BlockSpec((BR, D), blk)],
        out_shape=(jax.ShapeDtypeStruct((NROW, D), jnp.float32),
                   jax.ShapeDtypeStruct((NROW, D), jnp.float32)),
    )(p, cnts, x, w1r, b1, w2l)


def _fin_body(p_ref, c_ref, h1_ref, w2r, b2, wlin, blin, o_ref):
    cnt = c_ref[0] + c_ref[1]
    agg = (p_ref[0] + p_ref[1]) / jnp.maximum(cnt, 1.0)
    root = lax.dot_general(h1_ref[...], w2r[...], _DOT_T,
                           preferred_element_type=jnp.float32)
    h2 = jnp.maximum(agg + root + b2[...], 0.0)
    logits = lax.dot_general(h2, wlin[...], _DOT_T,
                             preferred_element_type=jnp.float32) + blin[...]
    col = lax.broadcasted_iota(jnp.int32, logits.shape, 1)
    logits = jnp.where(col < 2, logits, -1e30)
    m = jnp.max(logits, axis=1, keepdims=True)
    sh = logits - m
    lse = jnp.log(jnp.sum(jnp.exp(sh), axis=1, keepdims=True))
    o_ref[...] = lax.slice(sh - lse, (0, 0), (sh.shape[0], 2))


def _tc_fin(p, cnts, h1, w2r, b2, wlin, blin):
    blk = lambda i: (i, 0)
    blk3 = lambda i: (0, i, 0)
    whole = lambda i: (0, 0)
    return pl.pallas_call(
        _fin_body,
        grid=(NROW // BR,),
        in_specs=[pl.BlockSpec((NC, BR, D), blk3),
                  pl.BlockSpec((NC, BR, 1), blk3),
                  pl.BlockSpec((BR, D), blk),
                  pl.BlockSpec((D, D), whole), pl.BlockSpec((1, D), whole),
                  pl.BlockSpec((CW, D), whole), pl.BlockSpec((1, CW), whole)],
        out_specs=pl.BlockSpec((BR, 2), blk),
        out_shape=jax.ShapeDtypeStruct((N, 2), jnp.float32),
    )(p, cnts, h1, w2r, b2, wlin, blin)


def kernel(x, edge_index, W1l, b1, W1r, W2l, b2, W2r, Wlin, blin):
    f32 = jnp.float32
    src = edge_index[0].astype(jnp.int32)
    dst = edge_index[1].astype(jnp.int32)
    padv = N + (jnp.arange(EPAD - E, dtype=jnp.int32) % (NROW - N))
    src3 = jnp.concatenate([src, padv]).reshape(NW, CHUNKS, CSZ)
    dst3 = jnp.concatenate([dst, padv]).reshape(NW, CHUNKS, CSZ)
    zrow = jnp.zeros((NROW, D), f32)
    zcnt = jnp.zeros((NROW,), f32)
    ones = jnp.ones((CSZ,), f32)
    sc_agg_counts, sc_agg = _build_sc_kernels()

    x_pad = jnp.concatenate([x, jnp.zeros((NROW - N, D), f32)], axis=0)
    y1 = _tc_proj(x_pad, W1l)
    P1, C1 = sc_agg_counts(y1, src3, dst3, zrow, zcnt, ones)
    cnts = C1.reshape(NC, NROW, 1)
    h1, y2 = _tc_mid(P1, cnts, x_pad, W1r, b1.reshape(1, D), W2l)
    P2 = sc_agg(y2, src3, dst3, zrow)
    wlin_pad = jnp.concatenate([Wlin, jnp.zeros((CW - 2, D), f32)], axis=0)
    blin_pad = jnp.concatenate([blin, jnp.zeros((CW - 2,), f32)]).reshape(1, CW)
    return _tc_fin(P2, cnts, h1, W2r, b2.reshape(1, D), wlin_pad, blin_pad)
